# baseline (speedup 1.0000x reference)
import functools
import math

import jax
import jax.numpy as jnp
from jax import lax
from jax.experimental import pallas as pl
from jax.experimental.pallas import tpu as pltpu
from jax.experimental.pallas import tpu_sc as plsc

_V = 32000
_PAD = 0
_SMOOTH = 0.1
_CONF = 1.0 - _SMOOTH
_EPS = _SMOOTH / (_V - 2)
_CONST = (_V - 2) * _EPS * math.log(_EPS) + _CONF * math.log(_CONF)

_BR = 64
_L = 16
_CHUNK = 128


def _sc_gather_terms(xflat, t32, nw, bpw):
    mesh = plsc.VectorSubcoreMesh(core_axis_name="c", subcore_axis_name="s")
    nchunks = bpw // _CHUNK

    @functools.partial(
        pl.kernel,
        mesh=mesh,
        out_type=jax.ShapeDtypeStruct((nw * _L,), jnp.float32),
        scratch_types=[
            pltpu.VMEM((bpw,), jnp.int32),
            pltpu.VMEM((bpw,), jnp.int32),
            pltpu.VMEM((bpw,), jnp.int32),
            pltpu.VMEM((bpw,), jnp.float32),
            pltpu.VMEM((bpw,), jnp.float32),
            pltpu.VMEM((_L,), jnp.float32),
            pltpu.SemaphoreType.DMA,
            pltpu.SemaphoreType.DMA,
        ],
    )
    def k(x_hbm, t_hbm, out_hbm, t_v, idxt_v, idx0_v, g_v, x0_v, acc_v, s1, s2):
        wid = lax.axis_index("s") * 2 + lax.axis_index("c")
        base = wid * bpw
        pltpu.sync_copy(t_hbm.at[pl.ds(base, bpw)], t_v)
        lane = lax.iota(jnp.int32, _L)

        def build(j, carry):
            t = t_v[pl.ds(j * _L, _L)]
            i0 = ((base + j * _L) + lane) * _V
            idx0_v[pl.ds(j * _L, _L)] = i0
            idxt_v[pl.ds(j * _L, _L)] = i0 + t
            return carry

        lax.fori_loop(0, bpw // _L, build, 0)

        copies = []
        for c in range(nchunks):
            sl = pl.ds(c * _CHUNK, _CHUNK)
            copies.append(pltpu.async_copy(x_hbm.at[idxt_v.at[sl]], g_v.at[sl], s1))
            copies.append(pltpu.async_copy(x_hbm.at[idx0_v.at[sl]], x0_v.at[sl], s2))
        for cp in copies:
            cp.wait()

        def combine(j, acc):
            sl = pl.ds(j * _L, _L)
            t = t_v[sl]
            term = _CONST + _EPS * x0_v[sl] + (_EPS - _CONF) * g_v[sl]
            return acc + jnp.where(t != _PAD, term, 0.0)

        acc_v[...] = lax.fori_loop(0, bpw // _L, combine,
                                   jnp.zeros((_L,), jnp.float32))
        pltpu.sync_copy(acc_v, out_hbm.at[pl.ds(wid * _L, _L)])

    return k(xflat, t32)


def _tc_body(x_ref, t_ref, sc_ref, o_ref):
    step = pl.program_id(0)
    xb = x_ref[...]
    t = t_ref[...]
    srow = jnp.sum(xb, axis=1, keepdims=True)
    partial = -_EPS * jnp.sum(jnp.where(t != _PAD, srow, 0.0))

    @pl.when(step == 0)
    def _():
        o_ref[0, 0] = jnp.sum(sc_ref[...])

    o_ref[0, 0] += partial


def kernel(x, target):
    n, v = x.shape
    nw = 32
    bpw = n // nw
    t32 = target.astype(jnp.int32)
    sc_part = _sc_gather_terms(x.reshape(-1), t32, nw, bpw)
    out = pl.pallas_call(
        _tc_body,
        grid=(n // _BR,),
        in_specs=[
            pl.BlockSpec((_BR, v), lambda i: (i, 0)),
            pl.BlockSpec((_BR, 1), lambda i: (i, 0)),
            pl.BlockSpec((4, 128), lambda i: (0, 0)),
        ],
        out_specs=pl.BlockSpec(memory_space=pltpu.SMEM),
        out_shape=jax.ShapeDtypeStruct((1, 1), jnp.float32),
    )(x, t32.reshape(n, 1), sc_part.reshape(4, 128))
    return out[0, 0]


# speedup vs baseline: 2.9521x; 2.9521x over previous
import functools
import math

import jax
import jax.numpy as jnp
from jax import lax
from jax.experimental import pallas as pl
from jax.experimental.pallas import tpu as pltpu
from jax.experimental.pallas import tpu_sc as plsc

_V = 32000
_PAD = 0
_SMOOTH = 0.1
_CONF = 1.0 - _SMOOTH
_EPS = _SMOOTH / (_V - 2)
_CONST = (_V - 2) * _EPS * math.log(_EPS) + _CONF * math.log(_CONF)

_BR = 64
_L = 16
_NS = 16


def _tc_body(x_ref, t_ref, o_ref):
    xb = x_ref[...]
    t = t_ref[...]
    cols = lax.broadcasted_iota(jnp.int32, xb.shape, 1)
    srow = jnp.sum(xb, axis=1, keepdims=True)
    g = jnp.sum(jnp.where(cols == t, xb, 0.0), axis=1, keepdims=True)
    x0 = xb[:, 0:1]
    li = _CONST - _EPS * srow + _EPS * x0 + (_EPS - _CONF) * g
    o_ref[...] = jnp.where(t != _PAD, li, 0.0)


def _sc_reduce(terms):
    n = terms.shape[0]
    bpw = n // _NS
    mesh = plsc.VectorSubcoreMesh(core_axis_name="c", subcore_axis_name="s",
                                  num_cores=1)

    @functools.partial(
        pl.kernel,
        mesh=mesh,
        out_type=jax.ShapeDtypeStruct((_L,), jnp.float32),
        scratch_types=[
            pltpu.VMEM((bpw,), jnp.float32),
            pltpu.VMEM((_L,), jnp.float32),
            pltpu.VMEM((_NS * _L,), jnp.float32),
            pltpu.VMEM((2 * _L,), jnp.float32),
            pltpu.VMEM_SHARED((_NS * _L,), jnp.float32),
        ],
    )
    def k(terms_hbm, out_hbm, buf_v, st_v, all_v, win_v, shared):
        sid = lax.axis_index("s")
        pltpu.sync_copy(terms_hbm.at[pl.ds(sid * bpw, bpw)], buf_v)

        def body(j, acc):
            return acc + buf_v[pl.ds(j * _L, _L)]

        acc = lax.fori_loop(0, bpw // _L, body, jnp.zeros((_L,), jnp.float32))
        st_v[...] = acc
        pltpu.sync_copy(st_v, shared.at[pl.ds(sid * _L, _L)])
        plsc.subcore_barrier()

        @pl.when(sid == 0)
        def _():
            pltpu.sync_copy(shared.at[pl.ds(0, _NS * _L)], all_v)

            def body2(j, acc2):
                return acc2 + all_v[pl.ds(j * _L, _L)]

            tot = lax.fori_loop(0, _NS, body2, jnp.zeros((_L,), jnp.float32))
            win_v[pl.ds(0, _L)] = tot
            win_v[pl.ds(_L, _L)] = jnp.zeros((_L,), jnp.float32)
            s = tot
            for j in range(1, _L):
                s = s + win_v[pl.ds(j, _L)]
            st_v[...] = s
            pltpu.sync_copy(st_v, out_hbm)

    return k(terms)


def kernel(x, target):
    n, v = x.shape
    t2 = target.astype(jnp.int32).reshape(n, 1)
    terms = pl.pallas_call(
        _tc_body,
        grid=(n // _BR,),
        in_specs=[
            pl.BlockSpec((_BR, v), lambda i: (i, 0)),
            pl.BlockSpec((_BR, 1), lambda i: (i, 0)),
        ],
        out_specs=pl.BlockSpec((_BR, 1), lambda i: (i, 0)),
        out_shape=jax.ShapeDtypeStruct((n, 1), jnp.float32),
    )(x, t2)
    out = _sc_reduce(terms.reshape(n))
    return out[0]


# speedup vs baseline: 3.1038x; 1.0514x over previous
import functools
import math

import jax
import jax.numpy as jnp
from jax import lax
from jax.experimental import pallas as pl
from jax.experimental.pallas import tpu as pltpu
from jax.experimental.pallas import tpu_sc as plsc

_V = 32000
_PAD = 0
_SMOOTH = 0.1
_CONF = 1.0 - _SMOOTH
_EPS = _SMOOTH / (_V - 2)
_CONST = (_V - 2) * _EPS * math.log(_EPS) + _CONF * math.log(_CONF)

_BR = 128
_L = 16
_NS = 16


def _tc_body(x_ref, t_ref, o_ref):
    xb = x_ref[...]
    t = t_ref[...]
    cols = lax.broadcasted_iota(jnp.int32, xb.shape, 1)
    srow = jnp.sum(xb, axis=1, keepdims=True)
    g = jnp.sum(jnp.where(cols == t, xb, 0.0), axis=1, keepdims=True)
    x0 = xb[:, 0:1]
    li = _CONST - _EPS * srow + _EPS * x0 + (_EPS - _CONF) * g
    o_ref[...] = jnp.where(t != _PAD, li, 0.0)


def _sc_reduce(terms):
    n = terms.shape[0]
    bpw = n // _NS
    mesh = plsc.VectorSubcoreMesh(core_axis_name="c", subcore_axis_name="s",
                                  num_cores=1)

    @functools.partial(
        pl.kernel,
        mesh=mesh,
        out_type=jax.ShapeDtypeStruct((_L,), jnp.float32),
        scratch_types=[
            pltpu.VMEM((bpw,), jnp.float32),
            pltpu.VMEM((_L,), jnp.float32),
            pltpu.VMEM((_NS * _L,), jnp.float32),
            pltpu.VMEM((2 * _L,), jnp.float32),
            pltpu.VMEM_SHARED((_NS * _L,), jnp.float32),
        ],
    )
    def k(terms_hbm, out_hbm, buf_v, st_v, all_v, win_v, shared):
        sid = lax.axis_index("s")
        pltpu.sync_copy(terms_hbm.at[pl.ds(sid * bpw, bpw)], buf_v)

        def body(j, acc):
            return acc + buf_v[pl.ds(j * _L, _L)]

        acc = lax.fori_loop(0, bpw // _L, body, jnp.zeros((_L,), jnp.float32))
        st_v[...] = acc
        pltpu.sync_copy(st_v, shared.at[pl.ds(sid * _L, _L)])
        plsc.subcore_barrier()

        @pl.when(sid == 0)
        def _():
            pltpu.sync_copy(shared.at[pl.ds(0, _NS * _L)], all_v)

            def body2(j, acc2):
                return acc2 + all_v[pl.ds(j * _L, _L)]

            tot = lax.fori_loop(0, _NS, body2, jnp.zeros((_L,), jnp.float32))
            win_v[pl.ds(0, _L)] = tot
            win_v[pl.ds(_L, _L)] = jnp.zeros((_L,), jnp.float32)
            s = tot
            for j in range(1, _L):
                s = s + win_v[pl.ds(j, _L)]
            st_v[...] = s
            pltpu.sync_copy(st_v, out_hbm)

    return k(terms)


def kernel(x, target):
    n, v = x.shape
    t2 = target.astype(jnp.int32).reshape(n, 1)
    terms = pl.pallas_call(
        _tc_body,
        grid=(n // _BR,),
        in_specs=[
            pl.BlockSpec((_BR, v), lambda i: (i, 0)),
            pl.BlockSpec((_BR, 1), lambda i: (i, 0)),
        ],
        out_specs=pl.BlockSpec((_BR, 1), lambda i: (i, 0)),
        out_shape=jax.ShapeDtypeStruct((n, 1), jnp.float32),
    )(x, t2)
    out = _sc_reduce(terms.reshape(n))
    return out[0]


# speedup vs baseline: 3.1420x; 1.0123x over previous
import functools
import math

import jax
import jax.numpy as jnp
from jax import lax
from jax.experimental import pallas as pl
from jax.experimental.pallas import tpu as pltpu
from jax.experimental.pallas import tpu_sc as plsc

_V = 32000
_PAD = 0
_SMOOTH = 0.1
_CONF = 1.0 - _SMOOTH
_EPS = _SMOOTH / (_V - 2)
_CONST = (_V - 2) * _EPS * math.log(_EPS) + _CONF * math.log(_CONF)

_BR = 128
_L = 16
_NS = 16


def _tc_body(x_ref, t_ref, o_ref):
    xb = x_ref[...]
    t = t_ref[...]
    cols = lax.broadcasted_iota(jnp.int32, xb.shape, 1)
    srow = jnp.sum(xb, axis=1, keepdims=True)
    g = jnp.sum(jnp.where(cols == t, xb, 0.0), axis=1, keepdims=True)
    x0 = xb[:, 0:1]
    li = _CONST - _EPS * srow + _EPS * x0 + (_EPS - _CONF) * g
    o_ref[...] = jnp.sum(jnp.where(t != _PAD, li, 0.0)).reshape(1, 1, 1)


def _sc_reduce(terms):
    n = terms.shape[0]
    mesh = plsc.VectorSubcoreMesh(core_axis_name="c", subcore_axis_name="s",
                                  num_cores=1)

    @functools.partial(
        pl.kernel,
        mesh=mesh,
        out_type=jax.ShapeDtypeStruct((_L,), jnp.float32),
        scratch_types=[
            pltpu.VMEM((n,), jnp.float32),
            pltpu.VMEM((_L,), jnp.float32),
            pltpu.VMEM((2 * _L,), jnp.float32),
        ],
    )
    def k(terms_hbm, out_hbm, buf_v, st_v, win_v):
        sid = lax.axis_index("s")

        @pl.when(sid == 0)
        def _():
            pltpu.sync_copy(terms_hbm.at[pl.ds(0, n)], buf_v)

            def body(j, acc):
                return acc + buf_v[pl.ds(j * _L, _L)]

            tot = lax.fori_loop(0, n // _L, body,
                                jnp.zeros((_L,), jnp.float32))
            win_v[pl.ds(0, _L)] = tot
            win_v[pl.ds(_L, _L)] = jnp.zeros((_L,), jnp.float32)
            s = tot
            for j in range(1, _L):
                s = s + win_v[pl.ds(j, _L)]
            st_v[...] = s
            pltpu.sync_copy(st_v, out_hbm)

    return k(terms)


def kernel(x, target):
    n, v = x.shape
    t2 = target.astype(jnp.int32).reshape(n, 1)
    terms = pl.pallas_call(
        _tc_body,
        grid=(n // _BR,),
        in_specs=[
            pl.BlockSpec((_BR, v), lambda i: (i, 0)),
            pl.BlockSpec((_BR, 1), lambda i: (i, 0)),
        ],
        out_specs=pl.BlockSpec((1, 1, 1), lambda i: (i, 0, 0)),
        out_shape=jax.ShapeDtypeStruct((n // _BR, 1, 1), jnp.float32),
    )(x, t2)
    out = _sc_reduce(terms.reshape(n // _BR))
    return out[0]
